# baseline (speedup 1.0000x reference)
import jax
import jax.numpy as jnp
import numpy as np
from jax.experimental import pallas as pl
from jax.experimental.pallas import tpu as pltpu

_B = 16
_T, _H, _W = 16, 32, 64
_N = _T * _H * _W
_D = 8
_E = 8
_HID = 128
_SIZES = {'t': _T, 'h': _H, 'w': _W}
_TILE = 4096
_TP = _TILE // 16
_NT = _N // _TILE
_G = _B * _NT


def _marginal_sched(key, ax):
    size = _SIZES[ax]
    conc = jnp.full((_B, size), 1.0, dtype=jnp.float32)
    lp = jnp.log(jax.random.dirichlet(key, conc) + 1e-20)
    if ax == 't':
        g = lp[:, :, None, None]
    elif ax == 'h':
        g = lp[:, None, :, None]
    else:
        g = lp[:, None, None, :]
    return jnp.broadcast_to(g, (_B, _T, _H, _W)).reshape(_B, _N)


def _schedule():
    key = jax.random.key(42)
    kg, kt, kh, kw, ku = jax.random.split(key, 5)
    u = jax.random.uniform(kg, (_B, _N), minval=1e-9, maxval=1.0)
    g = -jnp.log(-jnp.log(u))
    ws = (g + _marginal_sched(kt, 't') + _marginal_sched(kh, 'h')
          + _marginal_sched(kw, 'w'))
    strat = jnp.linspace(0.0, 1.0, _B)
    rates = (jax.random.uniform(ku, (1,)) + strat) % 1.0
    ks = jnp.clip((_N * rates).astype(jnp.int32), 1, _N - 1)
    i = jax.lax.bitcast_convert_type(ws, jnp.int32)
    s = jnp.where(i < 0, i ^ jnp.int32(0x7FFFFFFF), i)
    return s, ks


def _search_body(s_ref, ks_ref, lo_ref):
    s = s_ref[...]
    k = ks_ref[...]
    lo0 = jnp.full((_B, 1), -2**31, jnp.int32)
    hi0 = jnp.full((_B, 1), 2**31 - 1, jnp.int32)

    def body(_, carry):
        lo, hi = carry
        mid = (lo & hi) + ((lo ^ hi) >> 1)
        cnt = jnp.sum((s > mid).astype(jnp.int32), axis=1, keepdims=True)
        p = cnt >= k
        return jnp.where(p, mid, lo), jnp.where(p, hi, mid)

    lo, _ = jax.lax.fori_loop(0, 32, body, (lo0, hi0))
    lo_ref[...] = lo


def _find_thresholds(s, ks2):
    return pl.pallas_call(
        _search_body,
        out_shape=jax.ShapeDtypeStruct((_B, 1), jnp.int32),
    )(s, ks2)


def _main_body(tokp_ref, stq_ref, sv_ref, lo_ref, cb_ref, w1b_ref, w1rt_ref,
               b1t_ref, w2b_ref, b2t_ref, se_ref, q2_ref, rep_ref, rep2_ref,
               ens_ref, part_ref, vis_ref):
    b = pl.program_id(0) // _NT
    ohc = (jax.lax.broadcasted_iota(jnp.int32, (_B, 1), 0) == b)
    lo_b = jnp.sum(jnp.where(ohc, lo_ref[...], 0))
    cb_b = jnp.sum(jnp.where(ohc, cb_ref[...], 0.0))
    vis_ref[0] = (sv_ref[0] > lo_b).astype(jnp.float32)
    mq = (stq_ref[...] > lo_b).astype(jnp.float32)
    rio = jax.lax.broadcasted_iota(jnp.int32, (256, 16), 0)
    cio = jax.lax.broadcasted_iota(jnp.int32, (256, 16), 1)
    sel = (rio == cio * _B + b).astype(jnp.float32)
    mb = jnp.dot(mq, sel, preferred_element_type=jnp.float32)

    tokp = tokp_ref[0]
    t1 = jnp.dot(tokp, w1b_ref[...], preferred_element_type=jnp.float32)
    mrep1 = jnp.dot(mb, rep_ref[...], preferred_element_type=jnp.float32)
    hbig = jnp.maximum((t1 + w1rt_ref[...]) * mrep1 + b1t_ref[...], 0.0)
    h2 = jnp.reshape(hbig, (_TILE // 2, 256))
    pred2 = jnp.dot(h2, w2b_ref[...],
                    preferred_element_type=jnp.float32) + b2t_ref[...]
    ens_ref[0] = pred2
    m2rep = jnp.reshape(
        jnp.dot(mb, rep2_ref[...], preferred_element_type=jnp.float32),
        (_TILE // 2, 128))
    w2rep = (1.0 - m2rep) * cb_b
    wpred2 = pred2 * w2rep
    tokrep2 = jnp.reshape(
        jnp.dot(tokp, se_ref[...], preferred_element_type=jnp.float32),
        (_TILE // 2, 128))
    s1 = jnp.sum(jnp.abs(wpred2 - tokrep2 * w2rep))
    zw = jnp.dot(wpred2, q2_ref[...], preferred_element_type=jnp.float32)
    s2 = jnp.sum(jnp.abs(zw))
    part_ref[...] = jnp.broadcast_to(
        s1 * (1.0 / _E) - s2 * (1.0 / (_E * _E)), (1, 1, 1))


def _build_consts():
    q = np.zeros((_D * _E, 4 * _D * _E), np.float32)
    for si, sh in enumerate((1, 2, 3, 4)):
        scale = 0.5 if sh == 4 else 1.0
        for d in range(_D):
            for e in range(_E):
                col = si * 64 + d * _E + e
                q[d * _E + e, col] += scale
                q[d * _E + (e + sh) % _E, col] -= scale
    q2 = np.kron(np.eye(2, dtype=np.float32), q)
    se = np.zeros((128, 1024), np.float32)
    for slot in range(16):
        for d in range(_D):
            for e in range(_E):
                se[slot * _D + d, slot * 64 + d * _E + e] = 1.0
    rep = np.kron(np.eye(16, dtype=np.float32),
                  np.ones((1, _HID), np.float32))
    rep2 = np.kron(np.eye(16, dtype=np.float32),
                   np.ones((1, 64), np.float32))
    return jnp.asarray(q2), jnp.asarray(se), jnp.asarray(rep), \
        jnp.asarray(rep2)


def kernel(tokens, W1, b1, W2, b2):
    with jax.ensure_compile_time_eval():
        s, ks = _schedule()
        q2, se, rep, rep2 = _build_consts()
        stq = s.T.reshape(_N // 16, 256)
        sv = s.reshape(_B, _N // 128, 128)
        ks2 = ks.reshape(_B, 1)
        cb = (_N / ((_N - ks2).astype(jnp.float32)
                    * (_B * _N * _D))).astype(jnp.float32)

    lo = _find_thresholds(s, ks2)

    w1big = jnp.kron(jnp.eye(16, dtype=jnp.float32), W1[:_D])
    w1rt = jnp.tile(W1[_D], 16).reshape(1, 16 * _HID)
    b1t = jnp.tile(b1, 16).reshape(1, 16 * _HID)
    w2b = jnp.kron(jnp.eye(2, dtype=jnp.float32), W2)
    b2t = jnp.tile(b2, 2).reshape(1, 128)

    const = lambda i: (0, 0)
    ens, parts, vis = pl.pallas_call(
        _main_body,
        grid=(_G,),
        in_specs=[
            pl.BlockSpec((1, _TP, 128), lambda i: (i // _NT, i % _NT, 0)),
            pl.BlockSpec((_TP, 256), lambda i: (i % _NT, 0)),
            pl.BlockSpec((1, _TILE // 128, 128),
                         lambda i: (i // _NT, i % _NT, 0)),
            pl.BlockSpec((_B, 1), const),
            pl.BlockSpec((_B, 1), const),
            pl.BlockSpec((_HID, 16 * _HID), const),
            pl.BlockSpec((1, 16 * _HID), const),
            pl.BlockSpec((1, 16 * _HID), const),
            pl.BlockSpec((256, 128), const),
            pl.BlockSpec((1, 128), const),
            pl.BlockSpec((128, 1024), const),
            pl.BlockSpec((128, 512), const),
            pl.BlockSpec((16, 2048), const),
            pl.BlockSpec((16, 1024), const),
        ],
        out_specs=[
            pl.BlockSpec((1, _TILE // 2, 128),
                         lambda i: (i // _NT, i % _NT, 0)),
            pl.BlockSpec((1, 1, 1), lambda i: (i, 0, 0)),
            pl.BlockSpec((1, _TILE // 128, 128),
                         lambda i: (i // _NT, i % _NT, 0)),
        ],
        out_shape=[
            jax.ShapeDtypeStruct((_B, _N // 2, 128), jnp.float32),
            jax.ShapeDtypeStruct((_G, 1, 1), jnp.float32),
            jax.ShapeDtypeStruct((_B, _N // 128, 128), jnp.float32),
        ],
        compiler_params=pltpu.CompilerParams(
            dimension_semantics=("parallel",)),
    )(tokens.reshape(_B, _N // 16, 128), stq, sv, lo, cb, w1big, w1rt, b1t,
      w2b, b2t, se, q2, rep, rep2)

    loss = jnp.sum(parts)
    ensemble = ens.reshape(_B, _N, _D, _E)
    visible = vis.reshape(_B, _N, 1) > 0.5
    return (loss, ensemble, visible)


# speedup vs baseline: 2.1875x; 2.1875x over previous
import jax
import jax.numpy as jnp
import numpy as np
from jax.experimental import pallas as pl
from jax.experimental.pallas import tpu as pltpu

_B = 16
_T, _H, _W = 16, 32, 64
_N = _T * _H * _W
_D = 8
_E = 8
_HID = 128
_SIZES = {'t': _T, 'h': _H, 'w': _W}
_TILE = 4096
_NT = _N // _TILE
_G = _B * _NT


def _marginal_sched(key, ax):
    size = _SIZES[ax]
    conc = jnp.full((_B, size), 1.0, dtype=jnp.float32)
    lp = jnp.log(jax.random.dirichlet(key, conc) + 1e-20)
    if ax == 't':
        g = lp[:, :, None, None]
    elif ax == 'h':
        g = lp[:, None, :, None]
    else:
        g = lp[:, None, None, :]
    return jnp.broadcast_to(g, (_B, _T, _H, _W)).reshape(_B, _N)


def _schedule():
    key = jax.random.key(42)
    kg, kt, kh, kw, ku = jax.random.split(key, 5)
    u = jax.random.uniform(kg, (_B, _N), minval=1e-9, maxval=1.0)
    g = -jnp.log(-jnp.log(u))
    ws = (g + _marginal_sched(kt, 't') + _marginal_sched(kh, 'h')
          + _marginal_sched(kw, 'w'))
    strat = jnp.linspace(0.0, 1.0, _B)
    rates = (jax.random.uniform(ku, (1,)) + strat) % 1.0
    ks = jnp.clip((_N * rates).astype(jnp.int32), 1, _N - 1)
    i = jax.lax.bitcast_convert_type(ws, jnp.int32)
    s = jnp.where(i < 0, i ^ jnp.int32(0x7FFFFFFF), i)
    return s, ks


def _search_body(s_ref, ks_ref, lo_ref):
    s = s_ref[...]
    k = ks_ref[...]
    lo0 = jnp.full((_B, 1), -2**31, jnp.int32)
    hi0 = jnp.full((_B, 1), 2**31 - 1, jnp.int32)

    def body(_, carry):
        lo, hi = carry
        mid = (lo & hi) + ((lo ^ hi) >> 1)
        cnt = jnp.sum((s > mid).astype(jnp.int32), axis=1, keepdims=True)
        p = cnt >= k
        return jnp.where(p, mid, lo), jnp.where(p, hi, mid)

    lo, _ = jax.lax.fori_loop(0, 32, body, (lo0, hi0))
    lo_ref[...] = lo


def _find_thresholds(s, ks2):
    return pl.pallas_call(
        _search_body,
        out_shape=jax.ShapeDtypeStruct((_B, 1), jnp.int32),
    )(s, ks2)


def _main_body(tok_ref, st_ref, sv_ref, lo_ref, cb_ref, w1_ref, w1r_ref,
               b1_ref, w2_ref, b2_ref, s_ref, q_ref, ens_ref, part_ref,
               vis_ref):
    b = pl.program_id(0) // _NT
    ohc = (jax.lax.broadcasted_iota(jnp.int32, (_B, 1), 0) == b)
    lo_b = jnp.sum(jnp.where(ohc, lo_ref[...], 0))
    cb_b = jnp.sum(jnp.where(ohc, cb_ref[...], 0.0))
    vis_ref[0] = (sv_ref[0] > lo_b).astype(jnp.float32)
    oh = (jax.lax.broadcasted_iota(jnp.int32, (1, _B), 1) == b).astype(
        jnp.float32)
    m16 = (st_ref[...] > lo_b).astype(jnp.float32)
    m = jnp.sum(m16 * oh, axis=1, keepdims=True)
    w = (1.0 - m) * cb_b

    tok = tok_ref[0]
    t1 = jnp.dot(tok, w1_ref[...], preferred_element_type=jnp.float32)
    h = jnp.maximum((t1 + w1r_ref[...]) * m + b1_ref[...], 0.0)
    pred = jnp.dot(h, w2_ref[...],
                   preferred_element_type=jnp.float32) + b2_ref[...]
    ens_ref[0] = pred
    wpred = pred * w
    wtokrep = jnp.dot(tok * w, s_ref[...], preferred_element_type=jnp.float32)
    s1 = jnp.sum(jnp.abs(wpred - wtokrep))
    zw = jnp.dot(wpred, q_ref[...], preferred_element_type=jnp.float32)
    s2 = jnp.sum(jnp.abs(zw))
    part_ref[...] = jnp.broadcast_to(
        s1 * (1.0 / _E) - s2 * (1.0 / (_E * _E)), (1, 1, 1))


def _build_consts():
    s = np.zeros((_D, _D * _E), np.float32)
    for d in range(_D):
        s[d, d * _E:(d + 1) * _E] = 1.0
    q = np.zeros((_D * _E, 4 * _D * _E), np.float32)
    for si, sh in enumerate((1, 2, 3, 4)):
        scale = 0.5 if sh == 4 else 1.0
        for d in range(_D):
            for e in range(_E):
                col = si * 64 + d * _E + e
                q[d * _E + e, col] += scale
                q[d * _E + (e + sh) % _E, col] -= scale
    return jnp.asarray(s), jnp.asarray(q)


def kernel(tokens, W1, b1, W2, b2):
    with jax.ensure_compile_time_eval():
        s, ks = _schedule()
        smat, qmat = _build_consts()
        st = s.T
        sv = s.reshape(_B, _N // 128, 128)
        ks2 = ks.reshape(_B, 1)
        cb = (_N / ((_N - ks2).astype(jnp.float32)
                    * (_B * _N * _D))).astype(jnp.float32)

    lo = _find_thresholds(s, ks2)

    w1a = W1[:_D]
    w1r = W1[_D:_D + 1]
    b1r = b1.reshape(1, _HID)
    b2r = b2.reshape(1, _D * _E)

    const = lambda i: (0, 0)
    ens, parts, vis = pl.pallas_call(
        _main_body,
        grid=(_G,),
        in_specs=[
            pl.BlockSpec((1, _TILE, _D), lambda i: (i // _NT, i % _NT, 0)),
            pl.BlockSpec((_TILE, _B), lambda i: (i % _NT, 0)),
            pl.BlockSpec((1, _TILE // 128, 128),
                         lambda i: (i // _NT, i % _NT, 0)),
            pl.BlockSpec((_B, 1), const),
            pl.BlockSpec((_B, 1), const),
            pl.BlockSpec((_D, _HID), const),
            pl.BlockSpec((1, _HID), const),
            pl.BlockSpec((1, _HID), const),
            pl.BlockSpec((_HID, _D * _E), const),
            pl.BlockSpec((1, _D * _E), const),
            pl.BlockSpec((_D, _D * _E), const),
            pl.BlockSpec((_D * _E, 4 * _D * _E), const),
        ],
        out_specs=[
            pl.BlockSpec((1, _TILE, _D * _E),
                         lambda i: (i // _NT, i % _NT, 0)),
            pl.BlockSpec((1, 1, 1), lambda i: (i, 0, 0)),
            pl.BlockSpec((1, _TILE // 128, 128),
                         lambda i: (i // _NT, i % _NT, 0)),
        ],
        out_shape=[
            jax.ShapeDtypeStruct((_B, _N, _D * _E), jnp.float32),
            jax.ShapeDtypeStruct((_G, 1, 1), jnp.float32),
            jax.ShapeDtypeStruct((_B, _N // 128, 128), jnp.float32),
        ],
        compiler_params=pltpu.CompilerParams(
            dimension_semantics=("parallel",)),
    )(tokens.reshape(_B, _N, _D), st, sv, lo, cb, w1a, w1r, b1r, W2, b2r,
      smat, qmat)

    loss = jnp.sum(parts)
    ensemble = ens.reshape(_B, _N, _D, _E)
    visible = vis.reshape(_B, _N, 1) > 0.5
    return (loss, ensemble, visible)
